# baseline (speedup 1.0000x reference)
import functools

import jax
import jax.numpy as jnp
from jax import lax
from jax.experimental import pallas as pl
from jax.experimental.pallas import tpu as pltpu
from jax.experimental.pallas import tpu_sc as plsc

N = 10000
E = 320000
D_IN = 128
H = 128
OUT = 64
L = 3

NC = 2
NS = 16
NW = NC * NS
EPW = E // NW
CHUNK = 40
NCHUNK = EPW // CHUNK
NBUF = 5
GDEPTH = 4
NOUTER = NCHUNK // NBUF
RPT = 624
ZROWS = 104
REM0 = NS * RPT
REM = N - REM0
DEG_W = 128


@functools.cache
def _sc_kernels():
    mesh = plsc.VectorSubcoreMesh(core_axis_name="c", subcore_axis_name="s")

    @functools.partial(
        pl.kernel,
        mesh=mesh,
        out_type=jax.ShapeDtypeStruct((NC, N, H), jnp.float32),
        scratch_types=[pltpu.VMEM((3, NBUF, 2, CHUNK), jnp.int32)]
          + [pltpu.VMEM((CHUNK, H), jnp.float32)] * NBUF
          + [pltpu.SemaphoreType.DMA]
          + [pltpu.SemaphoreType.DMA] * (2 * NBUF)
          + [pltpu.VMEM_SHARED((N, H), jnp.float32)],
    )
    def sc_agg(h_hbm, e_hbm, zero_hbm, out_hbm, idx_t, *ring):
        rows = ring[:NBUF]
        sem_t = ring[NBUF]
        sem_g = ring[NBUF + 1:2 * NBUF + 1]
        sem_s = ring[2 * NBUF + 1:3 * NBUF + 1]
        acc_sh = ring[3 * NBUF + 1]
        c = lax.axis_index("c")
        s = lax.axis_index("s")
        wid = s * NC + c
        row0 = s * RPT

        pltpu.sync_copy(zero_hbm, acc_sh.at[pl.ds(row0, RPT)])

        @pl.when(s == NS - 1)
        def _():
            pltpu.sync_copy(zero_hbm.at[pl.ds(0, REM)],
                            acc_sh.at[pl.ds(REM0, REM)])

        pltpu.sync_copy(e_hbm.at[wid, 0], idx_t.at[0])
        pltpu.async_copy(e_hbm.at[wid, 1], idx_t.at[1], sem_t)
        plsc.subcore_barrier()

        for b in range(GDEPTH):
            pltpu.async_copy(h_hbm.at[idx_t.at[0, b, 0]], rows[b], sem_g[b])

        def outer(g, carry):
            k0 = lax.rem(g, 3)
            k1 = lax.rem(g + 1, 3)
            k2 = lax.rem(g + 2, 3)
            for b in range(NBUF):
                jj = NBUF * g + b
                if b == 0:
                    @pl.when(g < NOUTER - 1)
                    def _():
                        pltpu.make_async_copy(e_hbm.at[wid, 0],
                                              idx_t.at[k1], sem_t).wait()
                pltpu.make_async_copy(h_hbm.at[pl.ds(0, CHUNK)], rows[b],
                                      sem_g[b]).wait()
                pltpu.async_copy(rows[b], acc_sh.at[idx_t.at[k0, b, 1]],
                                 sem_s[b], add=True)
                if b == 0:
                    @pl.when(g >= 1)
                    def _():
                        pltpu.make_async_copy(h_hbm.at[pl.ds(0, CHUNK)],
                                              rows[NBUF - 1],
                                              sem_s[NBUF - 1]).wait()
                else:
                    pltpu.make_async_copy(h_hbm.at[pl.ds(0, CHUNK)],
                                          rows[b - 1], sem_s[b - 1]).wait()
                nb = (b + GDEPTH) % NBUF
                gk = k0 if b == 0 else k1
                gr = (b + GDEPTH) % NBUF

                @pl.when(jj + GDEPTH < NCHUNK)
                def _():
                    pltpu.async_copy(h_hbm.at[idx_t.at[gk, gr, 0]],
                                     rows[nb], sem_g[nb])
                if b == 1:
                    @pl.when(g < NOUTER - 2)
                    def _():
                        pltpu.async_copy(e_hbm.at[wid, g + 2],
                                         idx_t.at[k2], sem_t)
            return carry

        lax.fori_loop(0, NOUTER, outer, 0)
        pltpu.make_async_copy(h_hbm.at[pl.ds(0, CHUNK)], rows[NBUF - 1],
                              sem_s[NBUF - 1]).wait()
        plsc.subcore_barrier()

        pltpu.sync_copy(acc_sh.at[pl.ds(row0, RPT)],
                        out_hbm.at[c, pl.ds(row0, RPT)])

        @pl.when(s == NS - 1)
        def _():
            pltpu.sync_copy(acc_sh.at[pl.ds(REM0, REM)],
                            out_hbm.at[c, pl.ds(REM0, REM)])

    return sc_agg


RB = 2000
GRID = N // RB


def _init_body(x_ref, w_ref, b_ref, dp_ref, h_ref, cs_ref, invdeg_ref):
    acc = jnp.dot(x_ref[...], w_ref[...],
                  preferred_element_type=jnp.float32) + b_ref[...]
    h = jnp.maximum(acc, 0.0)
    h_ref[...] = h

    @pl.when(pl.program_id(0) == 0)
    def _():
        cs_ref[...] = jnp.zeros_like(cs_ref)

    cs_ref[...] += jnp.sum(h, axis=0, keepdims=True)
    deg = jnp.sum(dp_ref[...], axis=(0, 2))
    invdeg_ref[...] = (1.0 / jnp.maximum(deg, 1.0))[:, None]


_tc_init = pl.pallas_call(
    _init_body,
    grid=(GRID,),
    in_specs=[
        pl.BlockSpec((RB, D_IN), lambda i: (i, 0)),
        pl.BlockSpec((D_IN, H), lambda i: (0, 0)),
        pl.BlockSpec((1, H), lambda i: (0, 0)),
        pl.BlockSpec((NC, RB, DEG_W), lambda i: (0, i, 0)),
    ],
    out_specs=[
        pl.BlockSpec((RB, H), lambda i: (i, 0)),
        pl.BlockSpec((1, H), lambda i: (0, 0)),
        pl.BlockSpec((RB, 1), lambda i: (i, 0)),
    ],
    out_shape=[
        jax.ShapeDtypeStruct((N, H), jnp.float32),
        jax.ShapeDtypeStruct((1, H), jnp.float32),
        jax.ShapeDtypeStruct((N, 1), jnp.float32),
    ],
)


def _layer_body(h_ref, p_ref, invdeg_ref, cs_ref, ws_ref, wn_ref, wt_ref,
                b_ref, out_ref, cs_out_ref):
    h = h_ref[...]
    agg = (p_ref[0] + p_ref[1]) * invdeg_ref[...]
    star = cs_ref[...] * (1.0 / N)
    acc = jnp.dot(h, ws_ref[...], preferred_element_type=jnp.float32)
    acc += jnp.dot(agg, wn_ref[...], preferred_element_type=jnp.float32)
    acc += jnp.dot(star, wt_ref[...],
                   preferred_element_type=jnp.float32) + b_ref[...]
    out = jnp.where(acc > 0, acc, 0.1 * acc)
    out_ref[...] = out

    @pl.when(pl.program_id(0) == 0)
    def _():
        cs_out_ref[...] = jnp.zeros_like(cs_out_ref)

    cs_out_ref[...] += jnp.sum(out, axis=0, keepdims=True)


_tc_layer = pl.pallas_call(
    _layer_body,
    grid=(GRID,),
    in_specs=[
        pl.BlockSpec((RB, H), lambda i: (i, 0)),
        pl.BlockSpec((NC, RB, H), lambda i: (0, i, 0)),
        pl.BlockSpec((RB, 1), lambda i: (i, 0)),
        pl.BlockSpec((1, H), lambda i: (0, 0)),
        pl.BlockSpec((H, H), lambda i: (0, 0)),
        pl.BlockSpec((H, H), lambda i: (0, 0)),
        pl.BlockSpec((H, H), lambda i: (0, 0)),
        pl.BlockSpec((1, H), lambda i: (0, 0)),
    ],
    out_specs=[
        pl.BlockSpec((RB, H), lambda i: (i, 0)),
        pl.BlockSpec((1, H), lambda i: (0, 0)),
    ],
    out_shape=[
        jax.ShapeDtypeStruct((N, H), jnp.float32),
        jax.ShapeDtypeStruct((1, H), jnp.float32),
    ],
)


def _final_body(h_ref, wh_ref, bh_ref, wo_ref, bo_ref, out_ref):
    h = h_ref[...]
    z = jnp.dot(h, wh_ref[...],
                preferred_element_type=jnp.float32) + bh_ref[...]
    z = jnp.maximum(z, 0.0) + h
    o = jnp.dot(z, wo_ref[...],
                preferred_element_type=jnp.float32) + bo_ref[...]
    m = jnp.max(o, axis=1, keepdims=True)
    ex = jnp.exp(o - m)
    lse = jnp.log(jnp.sum(ex, axis=1, keepdims=True)) + m
    out_ref[...] = o - lse


_tc_final = pl.pallas_call(
    _final_body,
    grid=(GRID,),
    in_specs=[
        pl.BlockSpec((RB, H), lambda i: (i, 0)),
        pl.BlockSpec((H, H), lambda i: (0, 0)),
        pl.BlockSpec((1, H), lambda i: (0, 0)),
        pl.BlockSpec((H, OUT), lambda i: (0, 0)),
        pl.BlockSpec((1, OUT), lambda i: (0, 0)),
    ],
    out_specs=pl.BlockSpec((RB, OUT), lambda i: (i, 0)),
    out_shape=jax.ShapeDtypeStruct((N, OUT), jnp.float32),
)


def kernel(x, edge_index, w_init, b_init, w_self, w_nbr, w_star, b_conv,
           w_h, b_h, w_out, b_out):
    sc_agg = _sc_kernels()
    src = edge_index[0].reshape(NW, NCHUNK, CHUNK)
    dst = edge_index[1].reshape(NW, NCHUNK, CHUNK)
    zero_h = jnp.zeros((RPT, H), jnp.float32)
    ones_h = jnp.full((N, H), 1.0 / H, jnp.float32)
    packed = jnp.stack([src.reshape(NW, NOUTER, NBUF, CHUNK),
                        dst.reshape(NW, NOUTER, NBUF, CHUNK)], axis=3)
    packed_deg = jnp.stack([dst.reshape(NW, NOUTER, NBUF, CHUNK),
                            dst.reshape(NW, NOUTER, NBUF, CHUNK)], axis=3)
    degparts = sc_agg(ones_h, packed_deg, zero_h)
    h, colsum, invdeg = _tc_init(x, w_init, b_init.reshape(1, H), degparts)
    for i in range(L):
        parts = sc_agg(h, packed, zero_h)
        h, colsum = _tc_layer(h, parts, invdeg, colsum, w_self[i], w_nbr[i],
                              w_star[i], b_conv[i].reshape(1, H))
    return _tc_final(h, w_h, b_h.reshape(1, H), w_out,
                     b_out.reshape(1, OUT))


# speedup vs baseline: 1.1138x; 1.1138x over previous
import functools

import jax
import jax.numpy as jnp
from jax import lax
from jax.experimental import pallas as pl
from jax.experimental.pallas import tpu as pltpu
from jax.experimental.pallas import tpu_sc as plsc

N = 10000
E = 320000
D_IN = 128
H = 128
OUT = 64
L = 3

NC = 2
NS = 16
NW = NC * NS
EPW = E // NW
CHUNK = 40
NCHUNK = EPW // CHUNK
NBUF = 6
GDEPTH = 4
DBUF = 6
RPT = 624
ZROWS = 104
REM0 = NS * RPT
REM = N - REM0
DEG_W = 128


@functools.cache
def _sc_kernels():
    mesh = plsc.VectorSubcoreMesh(core_axis_name="c", subcore_axis_name="s")

    @functools.partial(
        pl.kernel,
        mesh=mesh,
        out_type=jax.ShapeDtypeStruct((NC, N, H), jnp.float32),
        scratch_types=[pltpu.VMEM((CHUNK,), jnp.int32)] * NBUF
          + [pltpu.VMEM((CHUNK,), jnp.int32)] * NBUF
          + [pltpu.VMEM((CHUNK, H), jnp.float32)] * NBUF
          + [pltpu.SemaphoreType.DMA] * (2 * NBUF)
          + [pltpu.VMEM_SHARED((N, H), jnp.float32)],
    )
    def sc_agg(h_hbm, src_hbm, dst_hbm, zero_hbm, out_hbm, *ring):
        srci = ring[:NBUF]
        dsti = ring[NBUF:2 * NBUF]
        rows = ring[2 * NBUF:3 * NBUF]
        sem_i = ring[3 * NBUF:4 * NBUF]
        sem_g = ring[4 * NBUF:5 * NBUF]
        acc_sh = ring[5 * NBUF]
        c = lax.axis_index("c")
        s = lax.axis_index("s")
        wid = s * NC + c
        row0 = s * RPT

        pltpu.sync_copy(zero_hbm, acc_sh.at[pl.ds(row0, RPT)])

        @pl.when(s == NS - 1)
        def _():
            pltpu.sync_copy(zero_hbm.at[pl.ds(0, REM)],
                            acc_sh.at[pl.ds(REM0, REM)])

        plsc.subcore_barrier()

        def idx_start(j, b):
            pltpu.async_copy(src_hbm.at[wid, j], srci[b], sem_i[b])
            pltpu.async_copy(dst_hbm.at[wid, j], dsti[b], sem_i[b])

        def idx_wait(j, b):
            pltpu.make_async_copy(src_hbm.at[wid, j], srci[b],
                                  sem_i[b]).wait()
            pltpu.make_async_copy(dst_hbm.at[wid, j], dsti[b],
                                  sem_i[b]).wait()

        def gather_start(b):
            pltpu.async_copy(h_hbm.at[srci[b]], rows[b], sem_g[b])

        def gather_wait(b):
            pltpu.make_async_copy(h_hbm.at[pl.ds(0, CHUNK)], rows[b],
                                  sem_g[b]).wait()

        for b in range(NBUF):
            idx_start(b, b)
        for b in range(GDEPTH):
            idx_wait(b, b)
            gather_start(b)

        def outer(g, carry):
            for b in range(NBUF):
                jj = NBUF * g + b

                @pl.when(jj < NCHUNK)
                def _():
                    gather_wait(b)
                    pltpu.sync_copy(rows[b], acc_sh.at[dsti[b]], add=True)

                    @pl.when(jj + NBUF < NCHUNK)
                    def _():
                        idx_start(jj + NBUF, b)

                    nb = (b + GDEPTH) % NBUF

                    @pl.when(jj + GDEPTH < NCHUNK)
                    def _():
                        idx_wait(jj + GDEPTH, nb)
                        gather_start(nb)
            return carry

        lax.fori_loop(0, (NCHUNK + NBUF - 1) // NBUF, outer, 0)
        plsc.subcore_barrier()

        pltpu.sync_copy(acc_sh.at[pl.ds(row0, RPT)],
                        out_hbm.at[c, pl.ds(row0, RPT)])

        @pl.when(s == NS - 1)
        def _():
            pltpu.sync_copy(acc_sh.at[pl.ds(REM0, REM)],
                            out_hbm.at[c, pl.ds(REM0, REM)])

    @functools.partial(
        pl.kernel,
        mesh=mesh,
        out_type=jax.ShapeDtypeStruct((NC, N, H), jnp.float32),
        scratch_types=[pltpu.VMEM((CHUNK, H), jnp.float32)]
          + [pltpu.VMEM((CHUNK,), jnp.int32)] * DBUF
          + [pltpu.SemaphoreType.DMA] * (2 * DBUF)
          + [pltpu.VMEM_SHARED((N, H), jnp.float32)],
    )
    def sc_deg(dst_hbm, zero_hbm, out_hbm, ones_v, *ring):
        dsti = ring[:DBUF]
        sem_i = ring[DBUF:2 * DBUF]
        sem_s = ring[2 * DBUF:3 * DBUF]
        acc_sh = ring[3 * DBUF]
        c = lax.axis_index("c")
        s = lax.axis_index("s")
        wid = s * NC + c
        row0 = s * RPT

        def ofill(r, carry):
            for k in range(H // 16):
                ones_v[r, pl.ds(k * 16, 16)] = jnp.full((16,), 1.0 / H,
                                                        jnp.float32)
            return carry

        lax.fori_loop(0, CHUNK, ofill, 0)

        pltpu.sync_copy(zero_hbm, acc_sh.at[pl.ds(row0, RPT)])

        @pl.when(s == NS - 1)
        def _():
            pltpu.sync_copy(zero_hbm.at[pl.ds(0, REM)],
                            acc_sh.at[pl.ds(REM0, REM)])

        plsc.subcore_barrier()

        for b in range(DBUF):
            pltpu.async_copy(dst_hbm.at[wid, b], dsti[b], sem_i[b])

        def outer(g, carry):
            for b in range(DBUF):
                jj = DBUF * g + b

                @pl.when(jj < NCHUNK)
                def _():
                    pltpu.make_async_copy(dst_hbm.at[wid, 0], dsti[b],
                                          sem_i[b]).wait()
                    pltpu.async_copy(ones_v, acc_sh.at[dsti[b]], sem_s[b],
                                     add=True)
                    pb = (b + DBUF - 1) % DBUF

                    @pl.when(jj >= 1)
                    def _():
                        pltpu.make_async_copy(zero_hbm.at[pl.ds(0, CHUNK)],
                                              ones_v, sem_s[pb]).wait()

                    @pl.when(jj + DBUF - 1 < NCHUNK)
                    def _():
                        pltpu.async_copy(dst_hbm.at[wid, jj + DBUF - 1],
                                         dsti[pb], sem_i[pb])
            return carry

        lax.fori_loop(0, (NCHUNK + DBUF - 1) // DBUF, outer, 0)
        pltpu.make_async_copy(zero_hbm.at[pl.ds(0, CHUNK)], ones_v,
                              sem_s[(NCHUNK - 1) % DBUF]).wait()
        plsc.subcore_barrier()

        pltpu.sync_copy(acc_sh.at[pl.ds(row0, RPT)],
                        out_hbm.at[c, pl.ds(row0, RPT)])

        @pl.when(s == NS - 1)
        def _():
            pltpu.sync_copy(acc_sh.at[pl.ds(REM0, REM)],
                            out_hbm.at[c, pl.ds(REM0, REM)])

    return sc_agg, sc_deg


RB = 2000
GRID = N // RB


def _init_body(x_ref, w_ref, b_ref, dp_ref, h_ref, cs_ref, invdeg_ref):
    acc = jnp.dot(x_ref[...], w_ref[...],
                  preferred_element_type=jnp.float32) + b_ref[...]
    h = jnp.maximum(acc, 0.0)
    h_ref[...] = h

    @pl.when(pl.program_id(0) == 0)
    def _():
        cs_ref[...] = jnp.zeros_like(cs_ref)

    cs_ref[...] += jnp.sum(h, axis=0, keepdims=True)
    deg = jnp.sum(dp_ref[...], axis=(0, 2))
    invdeg_ref[...] = (1.0 / jnp.maximum(deg, 1.0))[:, None]


_tc_init = pl.pallas_call(
    _init_body,
    grid=(GRID,),
    in_specs=[
        pl.BlockSpec((RB, D_IN), lambda i: (i, 0)),
        pl.BlockSpec((D_IN, H), lambda i: (0, 0)),
        pl.BlockSpec((1, H), lambda i: (0, 0)),
        pl.BlockSpec((NC, RB, DEG_W), lambda i: (0, i, 0)),
    ],
    out_specs=[
        pl.BlockSpec((RB, H), lambda i: (i, 0)),
        pl.BlockSpec((1, H), lambda i: (0, 0)),
        pl.BlockSpec((RB, 1), lambda i: (i, 0)),
    ],
    out_shape=[
        jax.ShapeDtypeStruct((N, H), jnp.float32),
        jax.ShapeDtypeStruct((1, H), jnp.float32),
        jax.ShapeDtypeStruct((N, 1), jnp.float32),
    ],
)


def _layer_body(h_ref, p_ref, invdeg_ref, cs_ref, ws_ref, wn_ref, wt_ref,
                b_ref, out_ref, cs_out_ref):
    h = h_ref[...]
    agg = (p_ref[0] + p_ref[1]) * invdeg_ref[...]
    star = cs_ref[...] * (1.0 / N)
    acc = jnp.dot(h, ws_ref[...], preferred_element_type=jnp.float32)
    acc += jnp.dot(agg, wn_ref[...], preferred_element_type=jnp.float32)
    acc += jnp.dot(star, wt_ref[...],
                   preferred_element_type=jnp.float32) + b_ref[...]
    out = jnp.where(acc > 0, acc, 0.1 * acc)
    out_ref[...] = out

    @pl.when(pl.program_id(0) == 0)
    def _():
        cs_out_ref[...] = jnp.zeros_like(cs_out_ref)

    cs_out_ref[...] += jnp.sum(out, axis=0, keepdims=True)


_tc_layer = pl.pallas_call(
    _layer_body,
    grid=(GRID,),
    in_specs=[
        pl.BlockSpec((RB, H), lambda i: (i, 0)),
        pl.BlockSpec((NC, RB, H), lambda i: (0, i, 0)),
        pl.BlockSpec((RB, 1), lambda i: (i, 0)),
        pl.BlockSpec((1, H), lambda i: (0, 0)),
        pl.BlockSpec((H, H), lambda i: (0, 0)),
        pl.BlockSpec((H, H), lambda i: (0, 0)),
        pl.BlockSpec((H, H), lambda i: (0, 0)),
        pl.BlockSpec((1, H), lambda i: (0, 0)),
    ],
    out_specs=[
        pl.BlockSpec((RB, H), lambda i: (i, 0)),
        pl.BlockSpec((1, H), lambda i: (0, 0)),
    ],
    out_shape=[
        jax.ShapeDtypeStruct((N, H), jnp.float32),
        jax.ShapeDtypeStruct((1, H), jnp.float32),
    ],
)


def _final_body(h_ref, wh_ref, bh_ref, wo_ref, bo_ref, out_ref):
    h = h_ref[...]
    z = jnp.dot(h, wh_ref[...],
                preferred_element_type=jnp.float32) + bh_ref[...]
    z = jnp.maximum(z, 0.0) + h
    o = jnp.dot(z, wo_ref[...],
                preferred_element_type=jnp.float32) + bo_ref[...]
    m = jnp.max(o, axis=1, keepdims=True)
    ex = jnp.exp(o - m)
    lse = jnp.log(jnp.sum(ex, axis=1, keepdims=True)) + m
    out_ref[...] = o - lse


_tc_final = pl.pallas_call(
    _final_body,
    grid=(GRID,),
    in_specs=[
        pl.BlockSpec((RB, H), lambda i: (i, 0)),
        pl.BlockSpec((H, H), lambda i: (0, 0)),
        pl.BlockSpec((1, H), lambda i: (0, 0)),
        pl.BlockSpec((H, OUT), lambda i: (0, 0)),
        pl.BlockSpec((1, OUT), lambda i: (0, 0)),
    ],
    out_specs=pl.BlockSpec((RB, OUT), lambda i: (i, 0)),
    out_shape=jax.ShapeDtypeStruct((N, OUT), jnp.float32),
)


def kernel(x, edge_index, w_init, b_init, w_self, w_nbr, w_star, b_conv,
           w_h, b_h, w_out, b_out):
    sc_agg, sc_deg = _sc_kernels()
    src = edge_index[0].reshape(NW, NCHUNK, CHUNK)
    dst = edge_index[1].reshape(NW, NCHUNK, CHUNK)
    zero_h = jnp.zeros((RPT, H), jnp.float32)
    degparts = sc_deg(dst, zero_h)
    h, colsum, invdeg = _tc_init(x, w_init, b_init.reshape(1, H), degparts)
    for i in range(L):
        parts = sc_agg(h, src, dst, zero_h)
        h, colsum = _tc_layer(h, parts, invdeg, colsum, w_self[i], w_nbr[i],
                              w_star[i], b_conv[i].reshape(1, H))
    return _tc_final(h, w_h, b_h.reshape(1, H), w_out,
                     b_out.reshape(1, OUT))


# speedup vs baseline: 1.1387x; 1.0224x over previous
import functools

import jax
import jax.numpy as jnp
from jax import lax
from jax.experimental import pallas as pl
from jax.experimental.pallas import tpu as pltpu
from jax.experimental.pallas import tpu_sc as plsc

N = 10000
E = 320000
D_IN = 128
H = 128
OUT = 64
L = 3

NC = 2
NS = 16
NW = NC * NS
EPW = E // NW
CHUNK = 40
NCHUNK = EPW // CHUNK
NBUF = 5
NOUTER = NCHUNK // NBUF
GDEPTH = 4
DBUF = 6
RPT = 624
ZROWS = 104
REM0 = NS * RPT
REM = N - REM0
DEG_W = 128


@functools.cache
def _sc_kernels():
    mesh = plsc.VectorSubcoreMesh(core_axis_name="c", subcore_axis_name="s")

    @functools.partial(
        pl.kernel,
        mesh=mesh,
        out_type=jax.ShapeDtypeStruct((NC, N, H), jnp.float32),
        scratch_types=[pltpu.VMEM((3, NBUF, 2, CHUNK), jnp.int32)]
          + [pltpu.VMEM((CHUNK, H), jnp.float32)] * NBUF
          + [pltpu.SemaphoreType.DMA]
          + [pltpu.SemaphoreType.DMA] * NBUF
          + [pltpu.VMEM_SHARED((N, H), jnp.float32)],
    )
    def sc_agg(h_hbm, e_hbm, zero_hbm, out_hbm, idx_t, *ring):
        rows = ring[:NBUF]
        sem_t = ring[NBUF]
        sem_g = ring[NBUF + 1:2 * NBUF + 1]
        acc_sh = ring[2 * NBUF + 1]
        c = lax.axis_index("c")
        s = lax.axis_index("s")
        wid = s * NC + c
        row0 = s * RPT

        pltpu.sync_copy(zero_hbm, acc_sh.at[pl.ds(row0, RPT)])

        @pl.when(s == NS - 1)
        def _():
            pltpu.sync_copy(zero_hbm.at[pl.ds(0, REM)],
                            acc_sh.at[pl.ds(REM0, REM)])

        pltpu.sync_copy(e_hbm.at[wid, 0], idx_t.at[0])
        pltpu.async_copy(e_hbm.at[wid, 1], idx_t.at[1], sem_t)
        plsc.subcore_barrier()

        for b in range(GDEPTH):
            pltpu.async_copy(h_hbm.at[idx_t.at[0, b, 0]], rows[b], sem_g[b])

        def outer(g, carry):
            k0 = lax.rem(g, 3)
            k1 = lax.rem(g + 1, 3)
            k2 = lax.rem(g + 2, 3)
            for b in range(NBUF):
                jj = NBUF * g + b
                if b == 0:
                    @pl.when(g < NOUTER - 1)
                    def _():
                        pltpu.make_async_copy(e_hbm.at[wid, 0],
                                              idx_t.at[k1], sem_t).wait()
                pltpu.make_async_copy(h_hbm.at[pl.ds(0, CHUNK)], rows[b],
                                      sem_g[b]).wait()
                pltpu.sync_copy(rows[b], acc_sh.at[idx_t.at[k0, b, 1]],
                                add=True)
                nb = (b + GDEPTH) % NBUF
                gk = k0 if b == 0 else k1
                gr = (b + GDEPTH) % NBUF

                @pl.when(jj + GDEPTH < NCHUNK)
                def _():
                    pltpu.async_copy(h_hbm.at[idx_t.at[gk, gr, 0]],
                                     rows[nb], sem_g[nb])
                if b == 1:
                    @pl.when(g < NOUTER - 2)
                    def _():
                        pltpu.async_copy(e_hbm.at[wid, g + 2],
                                         idx_t.at[k2], sem_t)
            return carry

        lax.fori_loop(0, NOUTER, outer, 0)
        plsc.subcore_barrier()

        pltpu.sync_copy(acc_sh.at[pl.ds(row0, RPT)],
                        out_hbm.at[c, pl.ds(row0, RPT)])

        @pl.when(s == NS - 1)
        def _():
            pltpu.sync_copy(acc_sh.at[pl.ds(REM0, REM)],
                            out_hbm.at[c, pl.ds(REM0, REM)])

    @functools.partial(
        pl.kernel,
        mesh=mesh,
        out_type=jax.ShapeDtypeStruct((NC, N, H), jnp.float32),
        scratch_types=[pltpu.VMEM((CHUNK, H), jnp.float32)]
          + [pltpu.VMEM((CHUNK,), jnp.int32)] * DBUF
          + [pltpu.SemaphoreType.DMA] * (2 * DBUF)
          + [pltpu.VMEM_SHARED((N, H), jnp.float32)],
    )
    def sc_deg(dst_hbm, zero_hbm, out_hbm, ones_v, *ring):
        dsti = ring[:DBUF]
        sem_i = ring[DBUF:2 * DBUF]
        sem_s = ring[2 * DBUF:3 * DBUF]
        acc_sh = ring[3 * DBUF]
        c = lax.axis_index("c")
        s = lax.axis_index("s")
        wid = s * NC + c
        row0 = s * RPT

        def ofill(r, carry):
            for k in range(H // 16):
                ones_v[r, pl.ds(k * 16, 16)] = jnp.full((16,), 1.0 / H,
                                                        jnp.float32)
            return carry

        lax.fori_loop(0, CHUNK, ofill, 0)

        pltpu.sync_copy(zero_hbm, acc_sh.at[pl.ds(row0, RPT)])

        @pl.when(s == NS - 1)
        def _():
            pltpu.sync_copy(zero_hbm.at[pl.ds(0, REM)],
                            acc_sh.at[pl.ds(REM0, REM)])

        plsc.subcore_barrier()

        for b in range(DBUF):
            pltpu.async_copy(dst_hbm.at[wid, b], dsti[b], sem_i[b])

        def outer(g, carry):
            for b in range(DBUF):
                jj = DBUF * g + b

                @pl.when(jj < NCHUNK)
                def _():
                    pltpu.make_async_copy(dst_hbm.at[wid, 0], dsti[b],
                                          sem_i[b]).wait()
                    pltpu.async_copy(ones_v, acc_sh.at[dsti[b]], sem_s[b],
                                     add=True)
                    pb = (b + DBUF - 1) % DBUF

                    @pl.when(jj >= 1)
                    def _():
                        pltpu.make_async_copy(zero_hbm.at[pl.ds(0, CHUNK)],
                                              ones_v, sem_s[pb]).wait()

                    @pl.when(jj + DBUF - 1 < NCHUNK)
                    def _():
                        pltpu.async_copy(dst_hbm.at[wid, jj + DBUF - 1],
                                         dsti[pb], sem_i[pb])
            return carry

        lax.fori_loop(0, (NCHUNK + DBUF - 1) // DBUF, outer, 0)
        pltpu.make_async_copy(zero_hbm.at[pl.ds(0, CHUNK)], ones_v,
                              sem_s[(NCHUNK - 1) % DBUF]).wait()
        plsc.subcore_barrier()

        pltpu.sync_copy(acc_sh.at[pl.ds(row0, RPT)],
                        out_hbm.at[c, pl.ds(row0, RPT)])

        @pl.when(s == NS - 1)
        def _():
            pltpu.sync_copy(acc_sh.at[pl.ds(REM0, REM)],
                            out_hbm.at[c, pl.ds(REM0, REM)])

    return sc_agg, sc_deg


RB = 2000
GRID = N // RB


def _init_body(x_ref, w_ref, b_ref, dp_ref, h_ref, cs_ref, invdeg_ref):
    acc = jnp.dot(x_ref[...], w_ref[...],
                  preferred_element_type=jnp.float32) + b_ref[...]
    h = jnp.maximum(acc, 0.0)
    h_ref[...] = h

    @pl.when(pl.program_id(0) == 0)
    def _():
        cs_ref[...] = jnp.zeros_like(cs_ref)

    cs_ref[...] += jnp.sum(h, axis=0, keepdims=True)
    deg = jnp.sum(dp_ref[...], axis=(0, 2))
    invdeg_ref[...] = (1.0 / jnp.maximum(deg, 1.0))[:, None]


_tc_init = pl.pallas_call(
    _init_body,
    grid=(GRID,),
    in_specs=[
        pl.BlockSpec((RB, D_IN), lambda i: (i, 0)),
        pl.BlockSpec((D_IN, H), lambda i: (0, 0)),
        pl.BlockSpec((1, H), lambda i: (0, 0)),
        pl.BlockSpec((NC, RB, DEG_W), lambda i: (0, i, 0)),
    ],
    out_specs=[
        pl.BlockSpec((RB, H), lambda i: (i, 0)),
        pl.BlockSpec((1, H), lambda i: (0, 0)),
        pl.BlockSpec((RB, 1), lambda i: (i, 0)),
    ],
    out_shape=[
        jax.ShapeDtypeStruct((N, H), jnp.float32),
        jax.ShapeDtypeStruct((1, H), jnp.float32),
        jax.ShapeDtypeStruct((N, 1), jnp.float32),
    ],
)


def _layer_body(h_ref, p_ref, invdeg_ref, cs_ref, ws_ref, wn_ref, wt_ref,
                b_ref, out_ref, cs_out_ref):
    h = h_ref[...]
    agg = (p_ref[0] + p_ref[1]) * invdeg_ref[...]
    star = cs_ref[...] * (1.0 / N)
    acc = jnp.dot(h, ws_ref[...], preferred_element_type=jnp.float32)
    acc += jnp.dot(agg, wn_ref[...], preferred_element_type=jnp.float32)
    acc += jnp.dot(star, wt_ref[...],
                   preferred_element_type=jnp.float32) + b_ref[...]
    out = jnp.where(acc > 0, acc, 0.1 * acc)
    out_ref[...] = out

    @pl.when(pl.program_id(0) == 0)
    def _():
        cs_out_ref[...] = jnp.zeros_like(cs_out_ref)

    cs_out_ref[...] += jnp.sum(out, axis=0, keepdims=True)


_tc_layer = pl.pallas_call(
    _layer_body,
    grid=(GRID,),
    in_specs=[
        pl.BlockSpec((RB, H), lambda i: (i, 0)),
        pl.BlockSpec((NC, RB, H), lambda i: (0, i, 0)),
        pl.BlockSpec((RB, 1), lambda i: (i, 0)),
        pl.BlockSpec((1, H), lambda i: (0, 0)),
        pl.BlockSpec((H, H), lambda i: (0, 0)),
        pl.BlockSpec((H, H), lambda i: (0, 0)),
        pl.BlockSpec((H, H), lambda i: (0, 0)),
        pl.BlockSpec((1, H), lambda i: (0, 0)),
    ],
    out_specs=[
        pl.BlockSpec((RB, H), lambda i: (i, 0)),
        pl.BlockSpec((1, H), lambda i: (0, 0)),
    ],
    out_shape=[
        jax.ShapeDtypeStruct((N, H), jnp.float32),
        jax.ShapeDtypeStruct((1, H), jnp.float32),
    ],
)


def _final_body(h_ref, wh_ref, bh_ref, wo_ref, bo_ref, out_ref):
    h = h_ref[...]
    z = jnp.dot(h, wh_ref[...],
                preferred_element_type=jnp.float32) + bh_ref[...]
    z = jnp.maximum(z, 0.0) + h
    o = jnp.dot(z, wo_ref[...],
                preferred_element_type=jnp.float32) + bo_ref[...]
    m = jnp.max(o, axis=1, keepdims=True)
    ex = jnp.exp(o - m)
    lse = jnp.log(jnp.sum(ex, axis=1, keepdims=True)) + m
    out_ref[...] = o - lse


_tc_final = pl.pallas_call(
    _final_body,
    grid=(GRID,),
    in_specs=[
        pl.BlockSpec((RB, H), lambda i: (i, 0)),
        pl.BlockSpec((H, H), lambda i: (0, 0)),
        pl.BlockSpec((1, H), lambda i: (0, 0)),
        pl.BlockSpec((H, OUT), lambda i: (0, 0)),
        pl.BlockSpec((1, OUT), lambda i: (0, 0)),
    ],
    out_specs=pl.BlockSpec((RB, OUT), lambda i: (i, 0)),
    out_shape=jax.ShapeDtypeStruct((N, OUT), jnp.float32),
)


def kernel(x, edge_index, w_init, b_init, w_self, w_nbr, w_star, b_conv,
           w_h, b_h, w_out, b_out):
    sc_agg, sc_deg = _sc_kernels()
    src = edge_index[0].reshape(NW, NCHUNK, CHUNK)
    dst = edge_index[1].reshape(NW, NCHUNK, CHUNK)
    zero_h = jnp.zeros((RPT, H), jnp.float32)
    packed = jnp.stack([src.reshape(NW, NOUTER, NBUF, CHUNK),
                        dst.reshape(NW, NOUTER, NBUF, CHUNK)], axis=3)
    degparts = sc_deg(dst, zero_h)
    h, colsum, invdeg = _tc_init(x, w_init, b_init.reshape(1, H), degparts)
    for i in range(L):
        parts = sc_agg(h, packed, zero_h)
        h, colsum = _tc_layer(h, parts, invdeg, colsum, w_self[i], w_nbr[i],
                              w_star[i], b_conv[i].reshape(1, H))
    return _tc_final(h, w_h, b_h.reshape(1, H), w_out,
                     b_out.reshape(1, OUT))
